# baseline (speedup 1.0000x reference)
import jax
import jax.numpy as jnp
from jax import lax
from jax.experimental import pallas as pl
from jax.experimental.pallas import tpu as pltpu
from jax.experimental.pallas import tpu_sc as plsc

N = 8192
C = 8
K = 512
G = N * C
NW = 32
GW = G // NW
CH = 64
SL = K // 16


def _sc_select(x_hbm, k_hbm, out_hbm, xbuf, kv, obuf):
    wid = lax.axis_index("s") * 2 + lax.axis_index("c")
    base = wid * GW
    pltpu.sync_copy(k_hbm, kv)
    lanes = lax.iota(jnp.int32, 16)

    def chunk_body(cc, _c):
        row0 = base + cc * CH
        pltpu.sync_copy(x_hbm.at[pl.ds(row0, CH)], xbuf)

        def prep_body(sl, _p):
            for gp in range(CH):
                v = xbuf[gp, pl.ds(sl * 16, 16)]
                b = lax.bitcast_convert_type(v, jnp.int32)
                key = jnp.where(b < 0, (~b) ^ jnp.int32(-2**31), b)
                xbuf[gp, pl.ds(sl * 16, 16)] = lax.bitcast_convert_type(key, jnp.float32)
            return 0

        lax.fori_loop(0, SL, prep_body, 0)

        def group_body(gp, ovec):
            kvec = kv[gp & 7]
            lo0 = jnp.broadcast_to(jnp.int32(-2**31), (16,))
            hi0 = jnp.broadcast_to(jnp.int32(2**31 - 1), (16,))

            def bis(_, carry):
                lo, hi = carry
                mid = (lo >> 1) + (hi >> 1) + (lo & hi & 1)
                acc = jnp.zeros((16,), jnp.int32)
                for sl in range(SL):
                    kk = lax.bitcast_convert_type(xbuf[gp, pl.ds(sl * 16, 16)], jnp.int32)
                    acc = acc + plsc.all_reduce_population_count(kk > mid)
                up = acc > kvec
                lo = jnp.where(up, mid + 1, lo)
                hi = jnp.where(up, hi, mid)
                return lo, hi

            lo, _ = lax.fori_loop(0, 32, bis, (lo0, hi0))
            a = lo

            macc = jnp.zeros((16,), jnp.int32)
            for sl in range(SL):
                kk = lax.bitcast_convert_type(xbuf[gp, pl.ds(sl * 16, 16)], jnp.int32)
                macc = macc + plsc.all_reduce_population_count(kk > a)
            target = kvec - macc

            sidx_vec = jnp.zeros((16,), jnp.int32)
            cum = jnp.zeros((16,), jnp.int32)
            for sl in range(SL):
                kk = lax.bitcast_convert_type(xbuf[gp, pl.ds(sl * 16, 16)], jnp.int32)
                eq = kk == a
                eqi = jnp.where(eq, 1, 0)
                excl = plsc.cumsum(eqi) - eqi
                hit = jnp.logical_and(eq, (excl + cum) == target)
                f = plsc.all_reduce_ffs(hit)
                sidx_vec = jnp.where(f < 16, sl * 16 + f, sidx_vec)
                cum = cum + plsc.all_reduce_population_count(eq)

            ovec = jnp.where(lanes == (gp & 15), sidx_vec, ovec)

            @pl.when((gp & 15) == 15)
            def _store():
                obuf[pl.ds((gp // 16) * 16, 16)] = ovec

            return ovec

        lax.fori_loop(0, CH, group_body, jnp.zeros((16,), jnp.int32))
        pltpu.sync_copy(obuf, out_hbm.at[pl.ds(row0, CH)])
        return 0

    lax.fori_loop(0, GW // CH, chunk_body, 0)


def _expand_kernel(i_ref, o_ref):
    idx = i_ref[...]
    lanes = jax.lax.broadcasted_iota(jnp.int32, (idx.shape[0], K), 1)
    o_ref[...] = jnp.where(lanes == idx, 1.0, 0.0).astype(jnp.float32)


def kernel(latents, k):
    x = latents.reshape(G, K)
    k16 = jnp.broadcast_to(k.astype(jnp.int32)[:, None], (8, 16))

    mesh = plsc.VectorSubcoreMesh(
        core_axis_name="c", subcore_axis_name="s", num_cores=2)
    sel = pl.kernel(
        _sc_select,
        mesh=mesh,
        compiler_params=pltpu.CompilerParams(needs_layout_passes=False),
        out_type=jax.ShapeDtypeStruct((G,), jnp.int32),
        scratch_types=[
            pltpu.VMEM((CH, K), jnp.float32),
            pltpu.VMEM((8, 16), jnp.int32),
            pltpu.VMEM((CH,), jnp.int32),
        ],
    )(x, k16)

    RE = 512
    out = pl.pallas_call(
        _expand_kernel,
        grid=(G // RE,),
        in_specs=[pl.BlockSpec((RE, 1), lambda i: (i, 0))],
        out_specs=pl.BlockSpec((RE, K), lambda i: (i, 0)),
        out_shape=jax.ShapeDtypeStruct((G, K), jnp.float32),
    )(sel.reshape(G, 1))
    return out.reshape(N, C * K)


# speedup vs baseline: 2.0530x; 2.0530x over previous
import jax
import jax.numpy as jnp
from jax import lax
from jax.experimental import pallas as pl
from jax.experimental.pallas import tpu as pltpu
from jax.experimental.pallas import tpu_sc as plsc

N = 8192
C = 8
K = 512
G = N * C
R = 1024
TC_ROWS = 45056
SC_ROWS = G - TC_ROWS
NW = 32
GW = SC_ROWS // NW
CH = 64
SL = K // 16


def _select_kernel(x_ref, k_ref, o_ref):
    x = x_ref[...]
    kk = k_ref[0]
    xt = x.T
    b = jax.lax.bitcast_convert_type(xt, jnp.int32)
    key = jnp.where(b < 0, (~b) ^ jnp.int32(-2**31), b)

    kf = kk.astype(jnp.float32)
    lo0 = jnp.full((1, R), -2**31, jnp.int32)
    hi0 = jnp.full((1, R), 2**31 - 1, jnp.int32)

    def body(_, carry):
        lo, hi = carry
        mid = (lo >> 1) + (hi >> 1) + (lo & hi & 1)
        gt = jnp.where(key > mid, 1.0, 0.0)
        cnt = jnp.sum(gt, axis=0, keepdims=True)
        go_up = cnt > kf
        lo = jnp.where(go_up, mid + 1, lo)
        hi = jnp.where(go_up, hi, mid)
        return lo, hi

    lo, _ = jax.lax.fori_loop(0, 32, body, (lo0, hi0))
    a = lo
    eq = key == a
    m = jnp.sum(jnp.where(key > a, 1.0, 0.0), axis=0, keepdims=True)
    ii = jax.lax.broadcasted_iota(jnp.int32, (K, K), 0)
    jj = jax.lax.broadcasted_iota(jnp.int32, (K, K), 1)
    tril = jnp.where(jj < ii, 1.0, 0.0).astype(jnp.bfloat16)
    eqf = jnp.where(eq, 1.0, 0.0).astype(jnp.bfloat16)
    t = jax.lax.dot(tril, eqf, preferred_element_type=jnp.float32)
    sel = jnp.logical_and(eq, t == (kf - m))
    o_ref[...] = jnp.where(sel, 1.0, 0.0).astype(jnp.float32).T


def _sc_select(x_hbm, k_hbm, out_hbm, xbuf, kv, obuf):
    wid = lax.axis_index("s") * 2 + lax.axis_index("c")
    base = wid * GW
    pltpu.sync_copy(k_hbm, kv)
    lanes = lax.iota(jnp.int32, 16)

    def chunk_body(cc, _c):
        row0 = base + cc * CH
        pltpu.sync_copy(x_hbm.at[pl.ds(row0, CH)], xbuf)

        def prep_body(sl, _p):
            for gp in range(CH):
                v = xbuf[gp, pl.ds(sl * 16, 16)]
                b = lax.bitcast_convert_type(v, jnp.int32)
                key = jnp.where(b < 0, (~b) ^ jnp.int32(-2**31), b)
                xbuf[gp, pl.ds(sl * 16, 16)] = lax.bitcast_convert_type(
                    key, jnp.float32)
            return 0

        lax.fori_loop(0, SL, prep_body, 0)

        def group_body(gp, ovec):
            kvec = kv[gp & 7]
            lo0 = jnp.broadcast_to(jnp.int32(-2**31), (16,))
            hi0 = jnp.broadcast_to(jnp.int32(2**31 - 1), (16,))

            def bis(_, carry):
                lo, hi = carry
                mid = (lo >> 1) + (hi >> 1) + (lo & hi & 1)
                acc = jnp.zeros((16,), jnp.int32)
                for sl in range(SL):
                    kk = lax.bitcast_convert_type(
                        xbuf[gp, pl.ds(sl * 16, 16)], jnp.int32)
                    acc = acc + plsc.all_reduce_population_count(kk > mid)
                up = acc > kvec
                lo = jnp.where(up, mid + 1, lo)
                hi = jnp.where(up, hi, mid)
                return lo, hi

            lo, _ = lax.fori_loop(0, 32, bis, (lo0, hi0))
            a = lo

            macc = jnp.zeros((16,), jnp.int32)
            for sl in range(SL):
                kk = lax.bitcast_convert_type(
                    xbuf[gp, pl.ds(sl * 16, 16)], jnp.int32)
                macc = macc + plsc.all_reduce_population_count(kk > a)
            target = kvec - macc

            sidx_vec = jnp.zeros((16,), jnp.int32)
            cum = jnp.zeros((16,), jnp.int32)
            for sl in range(SL):
                kk = lax.bitcast_convert_type(
                    xbuf[gp, pl.ds(sl * 16, 16)], jnp.int32)
                eq = kk == a
                eqi = jnp.where(eq, 1, 0)
                excl = plsc.cumsum(eqi) - eqi
                hit = jnp.logical_and(eq, (excl + cum) == target)
                f = plsc.all_reduce_ffs(hit)
                sidx_vec = jnp.where(f < 16, sl * 16 + f, sidx_vec)
                cum = cum + plsc.all_reduce_population_count(eq)

            ovec = jnp.where(lanes == (gp & 15), sidx_vec, ovec)

            @pl.when((gp & 15) == 15)
            def _store():
                obuf[pl.ds((gp // 16) * 16, 16)] = ovec

            return ovec

        lax.fori_loop(0, CH, group_body, jnp.zeros((16,), jnp.int32))
        pltpu.sync_copy(obuf, out_hbm.at[pl.ds(row0, CH)])
        return 0

    lax.fori_loop(0, GW // CH, chunk_body, 0)


def _expand_kernel(i_ref, o_ref):
    idx = i_ref[...]
    lanes = jax.lax.broadcasted_iota(jnp.int32, (idx.shape[0], K), 1)
    o_ref[...] = jnp.where(lanes == idx, 1.0, 0.0).astype(jnp.float32)


def kernel(latents, k):
    x = latents.reshape(G, K)
    ki = k.astype(jnp.int32)

    k16 = jnp.broadcast_to(ki[:, None], (8, 16))
    mesh = plsc.VectorSubcoreMesh(
        core_axis_name="c", subcore_axis_name="s", num_cores=2)
    sel = pl.kernel(
        _sc_select,
        mesh=mesh,
        compiler_params=pltpu.CompilerParams(needs_layout_passes=False),
        out_type=jax.ShapeDtypeStruct((SC_ROWS,), jnp.int32),
        scratch_types=[
            pltpu.VMEM((CH, K), jnp.float32),
            pltpu.VMEM((8, 16), jnp.int32),
            pltpu.VMEM((CH,), jnp.int32),
        ],
    )(x[TC_ROWS:], k16)

    k_rows = jnp.tile(ki, TC_ROWS // C).reshape(TC_ROWS // R, 1, R)
    out_tc = pl.pallas_call(
        _select_kernel,
        grid=(TC_ROWS // R,),
        in_specs=[
            pl.BlockSpec((R, K), lambda i: (i, 0)),
            pl.BlockSpec((1, 1, R), lambda i: (i, 0, 0)),
        ],
        out_specs=pl.BlockSpec((R, K), lambda i: (i, 0)),
        out_shape=jax.ShapeDtypeStruct((TC_ROWS, K), jnp.float32),
    )(x[:TC_ROWS], k_rows)

    RE = 512
    out_sc = pl.pallas_call(
        _expand_kernel,
        grid=(SC_ROWS // RE,),
        in_specs=[pl.BlockSpec((RE, 1), lambda i: (i, 0))],
        out_specs=pl.BlockSpec((RE, K), lambda i: (i, 0)),
        out_shape=jax.ShapeDtypeStruct((SC_ROWS, K), jnp.float32),
    )(sel.reshape(SC_ROWS, 1))
    out = jnp.concatenate([out_tc, out_sc], axis=0)
    return out.reshape(N, C * K)


# speedup vs baseline: 2.2182x; 1.0805x over previous
import jax
import jax.numpy as jnp
from jax import lax
from jax.experimental import pallas as pl
from jax.experimental.pallas import tpu as pltpu
from jax.experimental.pallas import tpu_sc as plsc

N = 8192
C = 8
K = 512
G = N * C
R = 1024
TC_ROWS = 45056
SC_ROWS = G - TC_ROWS
NW = 32
GW = SC_ROWS // NW
CH = 64
SL = K // 16


def _select_kernel(x_ref, k_ref, o_ref):
    x = x_ref[...]
    kk = k_ref[0]
    xt = x.T
    b = jax.lax.bitcast_convert_type(xt, jnp.int32)
    key = jnp.where(b < 0, (~b) ^ jnp.int32(-2**31), b)

    kf = kk.astype(jnp.float32)
    lo0 = jnp.full((1, R), -2**31, jnp.int32)
    hi0 = jnp.full((1, R), 2**31 - 1, jnp.int32)

    def body(_, carry):
        lo, hi = carry
        mid = (lo >> 1) + (hi >> 1) + (lo & hi & 1)
        gt = jnp.where(key > mid, 1.0, 0.0)
        cnt = jnp.sum(gt, axis=0, keepdims=True)
        go_up = cnt > kf
        lo = jnp.where(go_up, mid + 1, lo)
        hi = jnp.where(go_up, hi, mid)
        return lo, hi

    lo, _ = jax.lax.fori_loop(0, 32, body, (lo0, hi0))
    a = lo
    eq = key == a
    m = jnp.sum(jnp.where(key > a, 1.0, 0.0), axis=0, keepdims=True)
    ii = jax.lax.broadcasted_iota(jnp.int32, (K, K), 0)
    jj = jax.lax.broadcasted_iota(jnp.int32, (K, K), 1)
    tril = jnp.where(jj < ii, 1.0, 0.0).astype(jnp.bfloat16)
    eqf = jnp.where(eq, 1.0, 0.0).astype(jnp.bfloat16)
    t = jax.lax.dot(tril, eqf, preferred_element_type=jnp.float32)
    sel = jnp.logical_and(eq, t == (kf - m))
    o_ref[...] = jnp.where(sel, 1.0, 0.0).astype(jnp.float32).T


def _sc_select(x_hbm, k_hbm, out_hbm, xbuf, kv, obuf):
    wid = lax.axis_index("s") * 2 + lax.axis_index("c")
    base = TC_ROWS + wid * GW
    pltpu.sync_copy(k_hbm, kv)
    lanes = lax.iota(jnp.int32, 16)

    def chunk_body(cc, _c):
        row0 = base + cc * CH
        pltpu.sync_copy(x_hbm.at[pl.ds(row0, CH)], xbuf)

        def prep_body(sl, _p):
            for gp in range(CH):
                v = xbuf[gp, pl.ds(sl * 16, 16)]
                b = lax.bitcast_convert_type(v, jnp.int32)
                key = jnp.where(b < 0, (~b) ^ jnp.int32(-2**31), b)
                xbuf[gp, pl.ds(sl * 16, 16)] = lax.bitcast_convert_type(
                    key, jnp.float32)
            return 0

        lax.fori_loop(0, SL, prep_body, 0)

        def group_body(gp, ovec):
            kvec = kv[gp & 7]
            lo0 = jnp.broadcast_to(jnp.int32(-2**31), (16,))
            hi0 = jnp.broadcast_to(jnp.int32(2**31 - 1), (16,))

            def bis(_, carry):
                lo, hi = carry
                mid = (lo >> 1) + (hi >> 1) + (lo & hi & 1)
                acc = jnp.zeros((16,), jnp.int32)
                for sl in range(SL):
                    kk = lax.bitcast_convert_type(
                        xbuf[gp, pl.ds(sl * 16, 16)], jnp.int32)
                    acc = acc + plsc.all_reduce_population_count(kk > mid)
                up = acc > kvec
                lo = jnp.where(up, mid + 1, lo)
                hi = jnp.where(up, hi, mid)
                return lo, hi

            lo, _ = lax.fori_loop(0, 32, bis, (lo0, hi0))
            a = lo

            macc = jnp.zeros((16,), jnp.int32)
            for sl in range(SL):
                kk = lax.bitcast_convert_type(
                    xbuf[gp, pl.ds(sl * 16, 16)], jnp.int32)
                macc = macc + plsc.all_reduce_population_count(kk > a)
            target = kvec - macc

            sidx_vec = jnp.zeros((16,), jnp.int32)
            cum = jnp.zeros((16,), jnp.int32)
            for sl in range(SL):
                kk = lax.bitcast_convert_type(
                    xbuf[gp, pl.ds(sl * 16, 16)], jnp.int32)
                eq = kk == a
                eqi = jnp.where(eq, 1, 0)
                excl = plsc.cumsum(eqi) - eqi
                hit = jnp.logical_and(eq, (excl + cum) == target)
                f = plsc.all_reduce_ffs(hit)
                sidx_vec = jnp.where(f < 16, sl * 16 + f, sidx_vec)
                cum = cum + plsc.all_reduce_population_count(eq)

            ovec = jnp.where(lanes == (gp & 15), sidx_vec, ovec)

            @pl.when((gp & 15) == 15)
            def _store():
                obuf[pl.ds((gp // 16) * 16, 16)] = ovec

            return ovec

        lax.fori_loop(0, CH, group_body, jnp.zeros((16,), jnp.int32))
        pltpu.sync_copy(obuf, out_hbm.at[pl.ds(row0 - TC_ROWS, CH)])
        return 0

    lax.fori_loop(0, GW // CH, chunk_body, 0)


def _expand_kernel(i_ref, o_ref):
    idx = i_ref[...]
    lanes = jax.lax.broadcasted_iota(jnp.int32, (idx.shape[0], K), 1)
    o_ref[...] = jnp.where(lanes == idx, 1.0, 0.0).astype(jnp.float32)


def kernel(latents, k):
    x = latents.reshape(G, K)
    ki = k.astype(jnp.int32)

    k16 = jnp.broadcast_to(ki[:, None], (8, 16))
    mesh = plsc.VectorSubcoreMesh(
        core_axis_name="c", subcore_axis_name="s", num_cores=2)
    sel = pl.kernel(
        _sc_select,
        mesh=mesh,
        compiler_params=pltpu.CompilerParams(needs_layout_passes=False),
        out_type=jax.ShapeDtypeStruct((SC_ROWS,), jnp.int32),
        scratch_types=[
            pltpu.VMEM((CH, K), jnp.float32),
            pltpu.VMEM((8, 16), jnp.int32),
            pltpu.VMEM((CH,), jnp.int32),
        ],
    )(x, k16)

    k_rows = jnp.tile(ki, TC_ROWS // C).reshape(TC_ROWS // R, 1, R)
    out_tc = pl.pallas_call(
        _select_kernel,
        grid=(TC_ROWS // R,),
        in_specs=[
            pl.BlockSpec((R, K), lambda i: (i, 0)),
            pl.BlockSpec((1, 1, R), lambda i: (i, 0, 0)),
        ],
        out_specs=pl.BlockSpec((R, K), lambda i: (i, 0)),
        out_shape=jax.ShapeDtypeStruct((TC_ROWS, K), jnp.float32),
    )(x, k_rows)

    RE = 512
    out_sc = pl.pallas_call(
        _expand_kernel,
        grid=(SC_ROWS // RE,),
        in_specs=[pl.BlockSpec((RE, 1), lambda i: (i, 0))],
        out_specs=pl.BlockSpec((RE, K), lambda i: (i, 0)),
        out_shape=jax.ShapeDtypeStruct((SC_ROWS, K), jnp.float32),
    )(sel.reshape(SC_ROWS, 1))
    out = jnp.concatenate([out_tc, out_sc], axis=0)
    return out.reshape(N, C * K)


# speedup vs baseline: 2.5538x; 1.1513x over previous
import jax
import jax.numpy as jnp
from jax import lax
from jax.experimental import pallas as pl
from jax.experimental.pallas import tpu as pltpu
from jax.experimental.pallas import tpu_sc as plsc

N = 8192
C = 8
K = 512
G = N * C
R = 1024
TC_ROWS = 53248
SC_ROWS = G - TC_ROWS
NW = 32
GW = SC_ROWS // NW
CH = 64
SL = K // 16


def _select_kernel(x_ref, k_ref, o_ref):
    x = x_ref[...]
    kk = k_ref[0]
    xt = x.T
    b = jax.lax.bitcast_convert_type(xt, jnp.int32)
    key = jnp.where(b < 0, (~b) ^ jnp.int32(-2**31), b)

    kf = kk.astype(jnp.float32)
    lo0 = jnp.full((1, R), -2**31, jnp.int32)
    hi0 = jnp.full((1, R), 2**31 - 1, jnp.int32)

    def body(_, carry):
        lo, hi = carry
        mid = (lo >> 1) + (hi >> 1) + (lo & hi & 1)
        gt = jnp.where(key > mid, 1.0, 0.0)
        cnt = jnp.sum(gt, axis=0, keepdims=True)
        go_up = cnt > kf
        lo = jnp.where(go_up, mid + 1, lo)
        hi = jnp.where(go_up, hi, mid)
        return lo, hi

    lo, _ = jax.lax.fori_loop(0, 32, body, (lo0, hi0))
    a = lo
    eq = key == a
    m = jnp.sum(jnp.where(key > a, 1.0, 0.0), axis=0, keepdims=True)
    ii = jax.lax.broadcasted_iota(jnp.int32, (K, K), 0)
    jj = jax.lax.broadcasted_iota(jnp.int32, (K, K), 1)
    tril = jnp.where(jj < ii, 1.0, 0.0).astype(jnp.bfloat16)
    eqf = jnp.where(eq, 1.0, 0.0).astype(jnp.bfloat16)
    t = jax.lax.dot(tril, eqf, preferred_element_type=jnp.float32)
    sel = jnp.logical_and(eq, t == (kf - m))
    o_ref[...] = jnp.where(sel, 1.0, 0.0).astype(jnp.float32).T


def _sc_select(x_hbm, k_hbm, out_hbm, xbuf, kv, obuf):
    wid = lax.axis_index("s") * 2 + lax.axis_index("c")
    base = TC_ROWS + wid * GW
    pltpu.sync_copy(k_hbm, kv)
    lanes = lax.iota(jnp.int32, 16)

    def chunk_body(cc, _c):
        row0 = base + cc * CH
        pltpu.sync_copy(x_hbm.at[pl.ds(row0, CH)], xbuf)

        def prep_body(sl, _p):
            for gp in range(CH):
                v = xbuf[gp, pl.ds(sl * 16, 16)]
                b = lax.bitcast_convert_type(v, jnp.int32)
                key = jnp.where(b < 0, (~b) ^ jnp.int32(-2**31), b)
                xbuf[gp, pl.ds(sl * 16, 16)] = lax.bitcast_convert_type(
                    key, jnp.float32)
            return 0

        lax.fori_loop(0, SL, prep_body, 0)

        def group_body(gp, ovec):
            kvec = kv[gp & 7]
            lo0 = jnp.broadcast_to(jnp.int32(-2**31), (16,))
            hi0 = jnp.broadcast_to(jnp.int32(2**31 - 1), (16,))

            def bis(_, carry):
                lo, hi = carry
                mid = (lo >> 1) + (hi >> 1) + (lo & hi & 1)
                acc = jnp.zeros((16,), jnp.int32)
                for sl in range(SL):
                    kk = lax.bitcast_convert_type(
                        xbuf[gp, pl.ds(sl * 16, 16)], jnp.int32)
                    acc = acc + plsc.all_reduce_population_count(kk > mid)
                up = acc > kvec
                lo = jnp.where(up, mid + 1, lo)
                hi = jnp.where(up, hi, mid)
                return lo, hi

            lo, _ = lax.fori_loop(0, 32, bis, (lo0, hi0))
            a = lo

            macc = jnp.zeros((16,), jnp.int32)
            for sl in range(SL):
                kk = lax.bitcast_convert_type(
                    xbuf[gp, pl.ds(sl * 16, 16)], jnp.int32)
                macc = macc + plsc.all_reduce_population_count(kk > a)
            target = kvec - macc

            sidx_vec = jnp.zeros((16,), jnp.int32)
            cum = jnp.zeros((16,), jnp.int32)
            for sl in range(SL):
                kk = lax.bitcast_convert_type(
                    xbuf[gp, pl.ds(sl * 16, 16)], jnp.int32)
                eq = kk == a
                eqi = jnp.where(eq, 1, 0)
                excl = plsc.cumsum(eqi) - eqi
                hit = jnp.logical_and(eq, (excl + cum) == target)
                f = plsc.all_reduce_ffs(hit)
                sidx_vec = jnp.where(f < 16, sl * 16 + f, sidx_vec)
                cum = cum + plsc.all_reduce_population_count(eq)

            ovec = jnp.where(lanes == (gp & 15), sidx_vec, ovec)

            @pl.when((gp & 15) == 15)
            def _store():
                obuf[pl.ds((gp // 16) * 16, 16)] = ovec

            return ovec

        lax.fori_loop(0, CH, group_body, jnp.zeros((16,), jnp.int32))
        pltpu.sync_copy(obuf, out_hbm.at[pl.ds(row0 - TC_ROWS, CH)])
        return 0

    lax.fori_loop(0, GW // CH, chunk_body, 0)


def _expand_kernel(prev_ref, i_ref, o_ref):
    del prev_ref
    idx = i_ref[...]
    lanes = jax.lax.broadcasted_iota(jnp.int32, (idx.shape[0], K), 1)
    o_ref[...] = jnp.where(lanes == idx, 1.0, 0.0).astype(jnp.float32)


def kernel(latents, k):
    x = latents.reshape(G, K)
    ki = k.astype(jnp.int32)

    k16 = jnp.broadcast_to(ki[:, None], (8, 16))
    mesh = plsc.VectorSubcoreMesh(
        core_axis_name="c", subcore_axis_name="s", num_cores=2)
    sel = pl.kernel(
        _sc_select,
        mesh=mesh,
        compiler_params=pltpu.CompilerParams(needs_layout_passes=False),
        out_type=jax.ShapeDtypeStruct((SC_ROWS,), jnp.int32),
        scratch_types=[
            pltpu.VMEM((CH, K), jnp.float32),
            pltpu.VMEM((8, 16), jnp.int32),
            pltpu.VMEM((CH,), jnp.int32),
        ],
    )(x, k16)

    k_rows = jnp.tile(ki, TC_ROWS // C).reshape(TC_ROWS // R, 1, R)
    out_tc = pl.pallas_call(
        _select_kernel,
        grid=(TC_ROWS // R,),
        in_specs=[
            pl.BlockSpec((R, K), lambda i: (i, 0)),
            pl.BlockSpec((1, 1, R), lambda i: (i, 0, 0)),
        ],
        out_specs=pl.BlockSpec((R, K), lambda i: (i, 0)),
        out_shape=jax.ShapeDtypeStruct((G, K), jnp.float32),
    )(x, k_rows)

    RE = 512
    out = pl.pallas_call(
        _expand_kernel,
        grid=(SC_ROWS // RE,),
        in_specs=[
            pl.BlockSpec(memory_space=pl.ANY),
            pl.BlockSpec((RE, 1), lambda i: (i, 0)),
        ],
        out_specs=pl.BlockSpec((RE, K), lambda i: (i + TC_ROWS // RE, 0)),
        out_shape=jax.ShapeDtypeStruct((G, K), jnp.float32),
        input_output_aliases={0: 0},
    )(out_tc, sel.reshape(SC_ROWS, 1))
    return out.reshape(N, C * K)


# speedup vs baseline: 2.6531x; 1.0389x over previous
import jax
import jax.numpy as jnp
from jax import lax
from jax.experimental import pallas as pl
from jax.experimental.pallas import tpu as pltpu
from jax.experimental.pallas import tpu_sc as plsc

N = 8192
C = 8
K = 512
G = N * C
R = 1024
TC_ROWS = 49152
SC_ROWS = G - TC_ROWS
NW = 32
GW = SC_ROWS // NW
CH = 64
SL = K // 16


def _select_kernel(x_ref, k_ref, o_ref):
    x = x_ref[...]
    kk = k_ref[0]
    xt = x.T
    b = jax.lax.bitcast_convert_type(xt, jnp.int32)
    key = jnp.where(b < 0, (~b) ^ jnp.int32(-2**31), b)

    kf = kk.astype(jnp.float32)
    lo0 = jnp.full((1, R), -2**31, jnp.int32)
    hi0 = jnp.full((1, R), 2**31 - 1, jnp.int32)

    def body(_, carry):
        lo, hi = carry
        mid = (lo >> 1) + (hi >> 1) + (lo & hi & 1)
        gt = jnp.where(key > mid, 1.0, 0.0)
        cnt = jnp.sum(gt, axis=0, keepdims=True)
        go_up = cnt > kf
        lo = jnp.where(go_up, mid + 1, lo)
        hi = jnp.where(go_up, hi, mid)
        return lo, hi

    lo, _ = jax.lax.fori_loop(0, 32, body, (lo0, hi0))
    a = lo
    eq = key == a
    m = jnp.sum(jnp.where(key > a, 1.0, 0.0), axis=0, keepdims=True)
    ii = jax.lax.broadcasted_iota(jnp.int32, (K, K), 0)
    jj = jax.lax.broadcasted_iota(jnp.int32, (K, K), 1)
    tril = jnp.where(jj < ii, 1.0, 0.0).astype(jnp.bfloat16)
    eqf = jnp.where(eq, 1.0, 0.0).astype(jnp.bfloat16)
    t = jax.lax.dot(tril, eqf, preferred_element_type=jnp.float32)
    sel = jnp.logical_and(eq, t == (kf - m))
    o_ref[...] = jnp.where(sel, 1.0, 0.0).astype(jnp.float32).T


def _sc_select(x_hbm, k_hbm, out_hbm, xbuf, kv, obuf):
    wid = lax.axis_index("s") * 2 + lax.axis_index("c")
    base = TC_ROWS + wid * GW
    pltpu.sync_copy(k_hbm, kv)
    lanes = lax.iota(jnp.int32, 16)

    def chunk_body(cc, _c):
        row0 = base + cc * CH
        pltpu.sync_copy(x_hbm.at[pl.ds(row0, CH)], xbuf)

        def prep_body(sl, _p):
            for gp in range(CH):
                v = xbuf[gp, pl.ds(sl * 16, 16)]
                b = lax.bitcast_convert_type(v, jnp.int32)
                key = jnp.where(b < 0, (~b) ^ jnp.int32(-2**31), b)
                xbuf[gp, pl.ds(sl * 16, 16)] = lax.bitcast_convert_type(
                    key, jnp.float32)
            return 0

        lax.fori_loop(0, SL, prep_body, 0)

        def group_body(gp, ovec):
            kvec = kv[gp & 7]
            lo0 = jnp.broadcast_to(jnp.int32(-2**31), (16,))
            hi0 = jnp.broadcast_to(jnp.int32(2**31 - 1), (16,))

            def bis(_, carry):
                lo, hi = carry
                mid = (lo >> 1) + (hi >> 1) + (lo & hi & 1)
                acc = jnp.zeros((16,), jnp.int32)
                for sl in range(SL):
                    kk = lax.bitcast_convert_type(
                        xbuf[gp, pl.ds(sl * 16, 16)], jnp.int32)
                    acc = acc + plsc.all_reduce_population_count(kk > mid)
                up = acc > kvec
                lo = jnp.where(up, mid + 1, lo)
                hi = jnp.where(up, hi, mid)
                return lo, hi

            lo, _ = lax.fori_loop(0, 32, bis, (lo0, hi0))
            a = lo

            macc = jnp.zeros((16,), jnp.int32)
            for sl in range(SL):
                kk = lax.bitcast_convert_type(
                    xbuf[gp, pl.ds(sl * 16, 16)], jnp.int32)
                macc = macc + plsc.all_reduce_population_count(kk > a)
            target = kvec - macc

            sidx_vec = jnp.zeros((16,), jnp.int32)
            cum = jnp.zeros((16,), jnp.int32)
            for sl in range(SL):
                kk = lax.bitcast_convert_type(
                    xbuf[gp, pl.ds(sl * 16, 16)], jnp.int32)
                eq = kk == a
                eqi = jnp.where(eq, 1, 0)
                excl = plsc.cumsum(eqi) - eqi
                hit = jnp.logical_and(eq, (excl + cum) == target)
                f = plsc.all_reduce_ffs(hit)
                sidx_vec = jnp.where(f < 16, sl * 16 + f, sidx_vec)
                cum = cum + plsc.all_reduce_population_count(eq)

            ovec = jnp.where(lanes == (gp & 15), sidx_vec, ovec)

            @pl.when((gp & 15) == 15)
            def _store():
                obuf[pl.ds((gp // 16) * 16, 16)] = ovec

            return ovec

        lax.fori_loop(0, CH, group_body, jnp.zeros((16,), jnp.int32))
        pltpu.sync_copy(obuf, out_hbm.at[pl.ds(row0 - TC_ROWS, CH)])
        return 0

    lax.fori_loop(0, GW // CH, chunk_body, 0)


def _expand_kernel(prev_ref, i_ref, o_ref):
    del prev_ref
    idx = i_ref[...]
    lanes = jax.lax.broadcasted_iota(jnp.int32, (idx.shape[0], K), 1)
    o_ref[...] = jnp.where(lanes == idx, 1.0, 0.0).astype(jnp.float32)


def kernel(latents, k):
    x = latents.reshape(G, K)
    ki = k.astype(jnp.int32)

    k16 = jnp.broadcast_to(ki[:, None], (8, 16))
    mesh = plsc.VectorSubcoreMesh(
        core_axis_name="c", subcore_axis_name="s", num_cores=2)
    sel = pl.kernel(
        _sc_select,
        mesh=mesh,
        compiler_params=pltpu.CompilerParams(needs_layout_passes=False),
        out_type=jax.ShapeDtypeStruct((SC_ROWS,), jnp.int32),
        scratch_types=[
            pltpu.VMEM((CH, K), jnp.float32),
            pltpu.VMEM((8, 16), jnp.int32),
            pltpu.VMEM((CH,), jnp.int32),
        ],
    )(x, k16)

    k_rows = jnp.tile(ki, TC_ROWS // C).reshape(TC_ROWS // R, 1, R)
    out_tc = pl.pallas_call(
        _select_kernel,
        grid=(TC_ROWS // R,),
        in_specs=[
            pl.BlockSpec((R, K), lambda i: (i, 0)),
            pl.BlockSpec((1, 1, R), lambda i: (i, 0, 0)),
        ],
        out_specs=pl.BlockSpec((R, K), lambda i: (i, 0)),
        out_shape=jax.ShapeDtypeStruct((G, K), jnp.float32),
    )(x, k_rows)

    RE = 512
    out = pl.pallas_call(
        _expand_kernel,
        grid=(SC_ROWS // RE,),
        in_specs=[
            pl.BlockSpec(memory_space=pl.ANY),
            pl.BlockSpec((RE, 1), lambda i: (i, 0)),
        ],
        out_specs=pl.BlockSpec((RE, K), lambda i: (i + TC_ROWS // RE, 0)),
        out_shape=jax.ShapeDtypeStruct((G, K), jnp.float32),
        input_output_aliases={0: 0},
    )(out_tc, sel.reshape(SC_ROWS, 1))
    return out.reshape(N, C * K)
